# baseline (speedup 1.0000x reference)
import jax
import jax.numpy as jnp
from jax.experimental import pallas as pl
from jax.experimental.pallas import tpu as pltpu

N = 5000
NP = 5120
BI = 1024
BJ = 2048
SCORE_THR = 0.3
THIRD = 1.0 / 3.0


def _nms_body(k_ref, aT_ref, a8_ref, out_ref, fR_ref, fC_ref):
    i = pl.program_id(0)
    i0 = i * BI
    kk = k_ref[0]

    @pl.when(i == 0)
    def _features():
        cx = aT_ref[0:1, :]
        cy = aT_ref[1:2, :]
        w = aT_ref[3:4, :]
        l = aT_ref[4:5, :]
        ry = aT_ref[6:7, :]
        sc = aT_ref[7:8, :]
        c = jnp.abs(jnp.cos(ry))
        s = jnp.abs(jnp.sin(ry))
        hx = 0.5 * (w * c + l * s)
        hy = 0.5 * (w * s + l * c)
        x1 = cx - hx
        y1 = cy - hy
        x2 = cx + hx
        y2 = cy + hy
        a3 = ((x2 - x1) * (y2 - y1)) * THIRD
        rows = jnp.concatenate([x1, y1, x2, y2, a3, sc, sc, sc], axis=0)
        fR_ref[...] = rows
        fC_ref[...] = rows.T

    fc = fC_ref[pl.ds(i0, BI), :]
    ix1 = fc[:, 0:1]
    iy1 = fc[:, 1:2]
    ix2 = fc[:, 2:3]
    iy2 = fc[:, 3:4]
    ia3 = fc[:, 4:5]
    isc = fc[:, 5:6]

    @pl.when(i0 < kk)
    def _compute():
        irank = i0 + jax.lax.broadcasted_iota(jnp.int32, (BI, 1), 0)
        jlim = jnp.minimum(irank, kk)
        jiota = jax.lax.broadcasted_iota(jnp.int32, (1, BJ), 1)

        def tile(j0, masked, acc):
            jx1 = fR_ref[0:1, pl.ds(j0, BJ)]
            jy1 = fR_ref[1:2, pl.ds(j0, BJ)]
            jx2 = fR_ref[2:3, pl.ds(j0, BJ)]
            jy2 = fR_ref[3:4, pl.ds(j0, BJ)]
            ja3 = fR_ref[4:5, pl.ds(j0, BJ)]

            iw = jnp.maximum(jnp.minimum(ix2, jx2) - jnp.maximum(ix1, jx1),
                             0.0)
            ih = jnp.minimum(iy2, jy2) - jnp.maximum(iy1, jy1)
            marg = iw * ih - (ia3 + ja3)
            if masked:
                marg = jnp.where((j0 + jiota) < jlim, marg, -1.0)
            return jnp.maximum(acc, jnp.max(marg, axis=1, keepdims=True))

        nfull = jnp.minimum(i0, kk) // BJ
        acc0 = jnp.full((BI, 1), -1.0, dtype=jnp.float32)
        acc = jax.lax.fori_loop(
            0, nfull, lambda jj, a: tile(jj * BJ, False, a), acc0)
        nall = (jnp.minimum(i0 + BI, kk) + (BJ - 1)) // BJ
        acc = jax.lax.fori_loop(
            nfull, nall, lambda jj, a: tile(jj * BJ, True, a), acc)

        keepf = jnp.where((isc > SCORE_THR) & (acc <= 0.0), 1.0, 0.0)
        out_ref[...] = a8_ref[...] * keepf

    @pl.when(i0 >= kk)
    def _zeros():
        out_ref[...] = jnp.zeros((BI, 8), dtype=jnp.float32)


@jax.jit
def kernel(x, guided_anchors):
    pad = NP - N
    s = jax.nn.sigmoid(x)
    order = jnp.argsort(-s, stable=True)
    kcount = jnp.sum(s > SCORE_THR).astype(jnp.int32).reshape(1)

    a8 = jnp.concatenate([guided_anchors, s[:, None]], axis=1)[order]
    a8 = jnp.pad(a8, ((0, pad), (0, 0)))
    aT = a8.T

    out_sorted = pl.pallas_call(
        _nms_body,
        grid=(NP // BI,),
        in_specs=[
            pl.BlockSpec(memory_space=pltpu.SMEM),
            pl.BlockSpec((8, NP), lambda i: (0, 0)),
            pl.BlockSpec((BI, 8), lambda i: (i, 0)),
        ],
        out_specs=pl.BlockSpec((BI, 8), lambda i: (i, 0)),
        out_shape=jax.ShapeDtypeStruct((NP, 8), jnp.float32),
        scratch_shapes=[
            pltpu.VMEM((8, NP), jnp.float32),
            pltpu.VMEM((NP, 8), jnp.float32),
        ],
    )(kcount, aT, a8)

    return jnp.zeros((N, 8), jnp.float32).at[order].set(out_sorted[:N])


# speedup vs baseline: 1.3724x; 1.3724x over previous
import jax
import jax.numpy as jnp
from jax.experimental import pallas as pl
from jax.experimental.pallas import tpu as pltpu

N = 5000
NP = 5120
BI = 1024
BJ = 2048
SCORE_THR = 0.3
THIRD = 1.0 / 3.0


def _nms_body(k_ref, aT_ref, out_ref, fR_ref, fC_ref, raw_ref):
    i = pl.program_id(0)
    i0 = i * BI
    kk = k_ref[0]

    @pl.when(i == 0)
    def _features():
        cx = aT_ref[0:1, :]
        cy = aT_ref[1:2, :]
        w = aT_ref[3:4, :]
        l = aT_ref[4:5, :]
        ry = aT_ref[6:7, :]
        sc = aT_ref[7:8, :]
        c = jnp.abs(jnp.cos(ry))
        s = jnp.abs(jnp.sin(ry))
        hx = 0.5 * (w * c + l * s)
        hy = 0.5 * (w * s + l * c)
        x1 = cx - hx
        y1 = cy - hy
        x2 = cx + hx
        y2 = cy + hy
        a3 = ((x2 - x1) * (y2 - y1)) * THIRD
        rows = jnp.concatenate([x1, y1, x2, y2, a3, sc, sc, sc], axis=0)
        fR_ref[...] = rows
        fC_ref[...] = rows.T
        raw_ref[...] = aT_ref[...].T

    fc = fC_ref[pl.ds(i0, BI), :]
    ix1 = fc[:, 0:1]
    iy1 = fc[:, 1:2]
    ix2 = fc[:, 2:3]
    iy2 = fc[:, 3:4]
    ia3 = fc[:, 4:5]
    isc = fc[:, 5:6]

    @pl.when(i0 < kk)
    def _compute():
        irank = i0 + jax.lax.broadcasted_iota(jnp.int32, (BI, 1), 0)
        jlim = jnp.minimum(irank, kk)
        jiota = jax.lax.broadcasted_iota(jnp.int32, (1, BJ), 1)

        def tile(j0, masked, acc):
            jx1 = fR_ref[0:1, pl.ds(j0, BJ)]
            jy1 = fR_ref[1:2, pl.ds(j0, BJ)]
            jx2 = fR_ref[2:3, pl.ds(j0, BJ)]
            jy2 = fR_ref[3:4, pl.ds(j0, BJ)]
            ja3 = fR_ref[4:5, pl.ds(j0, BJ)]

            iw = jnp.maximum(jnp.minimum(ix2, jx2) - jnp.maximum(ix1, jx1),
                             0.0)
            ih = jnp.minimum(iy2, jy2) - jnp.maximum(iy1, jy1)
            marg = iw * ih - (ia3 + ja3)
            if masked:
                marg = jnp.where((j0 + jiota) < jlim, marg, -1.0)
            return jnp.maximum(acc, jnp.max(marg, axis=1, keepdims=True))

        nfull = jnp.minimum(i0, kk) // BJ
        acc0 = jnp.full((BI, 1), -1.0, dtype=jnp.float32)
        acc = jax.lax.fori_loop(
            0, nfull, lambda jj, a: tile(jj * BJ, False, a), acc0)
        nall = (jnp.minimum(i0 + BI, kk) + (BJ - 1)) // BJ
        acc = jax.lax.fori_loop(
            nfull, nall, lambda jj, a: tile(jj * BJ, True, a), acc)

        keepf = jnp.where((isc > SCORE_THR) & (acc <= 0.0), 1.0, 0.0)
        out_ref[...] = raw_ref[pl.ds(i0, BI), :] * keepf

    @pl.when(i0 >= kk)
    def _zeros():
        out_ref[...] = jnp.zeros((BI, 8), dtype=jnp.float32)


@jax.jit
def kernel(x, guided_anchors):
    pad = NP - N
    s = jax.nn.sigmoid(x)
    kcount = jnp.sum(s > SCORE_THR).astype(jnp.int32).reshape(1)

    idx = jnp.arange(N, dtype=jnp.int32)
    cols = [guided_anchors[:, c] for c in range(7)]
    srt = jax.lax.sort([-s] + cols + [s, idx], num_keys=1, is_stable=True)
    order = srt[9]
    aT = jnp.stack(srt[1:9], axis=0)
    aT = jnp.pad(aT, ((0, 0), (0, pad)))

    out_sorted = pl.pallas_call(
        _nms_body,
        grid=(NP // BI,),
        in_specs=[
            pl.BlockSpec(memory_space=pltpu.SMEM),
            pl.BlockSpec((8, NP), lambda i: (0, 0)),
        ],
        out_specs=pl.BlockSpec((BI, 8), lambda i: (i, 0)),
        out_shape=jax.ShapeDtypeStruct((NP, 8), jnp.float32),
        scratch_shapes=[
            pltpu.VMEM((8, NP), jnp.float32),
            pltpu.VMEM((NP, 8), jnp.float32),
            pltpu.VMEM((NP, 8), jnp.float32),
        ],
    )(kcount, aT)

    return jnp.zeros((N, 8), jnp.float32).at[order].set(out_sorted[:N])


# speedup vs baseline: 1.3990x; 1.0193x over previous
import jax
import jax.numpy as jnp
from jax.experimental import pallas as pl
from jax.experimental.pallas import tpu as pltpu

N = 5000
NP = 5120
BI = 1024
BJ = 1024
SCORE_THR = 0.3
THIRD = 1.0 / 3.0


def _nms_body(k_ref, aT_ref, keep_ref, fR_ref, fC_ref):
    i = pl.program_id(0)
    i0 = i * BI
    kk = k_ref[0]

    @pl.when(i == 0)
    def _features():
        cx = aT_ref[0:1, :]
        cy = aT_ref[1:2, :]
        w = aT_ref[3:4, :]
        l = aT_ref[4:5, :]
        ry = aT_ref[6:7, :]
        sc = aT_ref[7:8, :]
        c = jnp.abs(jnp.cos(ry))
        s = jnp.abs(jnp.sin(ry))
        hx = 0.5 * (w * c + l * s)
        hy = 0.5 * (w * s + l * c)
        x1 = cx - hx
        y1 = cy - hy
        x2 = cx + hx
        y2 = cy + hy
        a3 = ((x2 - x1) * (y2 - y1)) * THIRD
        rows = jnp.concatenate([x1, y1, x2, y2, a3, sc, sc, sc], axis=0)
        fR_ref[...] = rows
        fC_ref[...] = rows.T

    ix1 = fR_ref[0:1, pl.ds(i0, BI)]
    iy1 = fR_ref[1:2, pl.ds(i0, BI)]
    ix2 = fR_ref[2:3, pl.ds(i0, BI)]
    iy2 = fR_ref[3:4, pl.ds(i0, BI)]
    ia3 = fR_ref[4:5, pl.ds(i0, BI)]
    isc = fR_ref[5:6, pl.ds(i0, BI)]

    @pl.when(i0 < kk)
    def _compute():
        irank = i0 + jax.lax.broadcasted_iota(jnp.int32, (1, BI), 1)
        jlim = jnp.minimum(irank, kk)
        jiota = jax.lax.broadcasted_iota(jnp.int32, (BJ, 1), 0)

        def tile(j0, masked, acc):
            fj = fC_ref[pl.ds(j0, BJ), :]
            jx1 = fj[:, 0:1]
            jy1 = fj[:, 1:2]
            jx2 = fj[:, 2:3]
            jy2 = fj[:, 3:4]
            ja3 = fj[:, 4:5]

            iw = jnp.maximum(jnp.minimum(ix2, jx2) - jnp.maximum(ix1, jx1),
                             0.0)
            ih = jnp.minimum(iy2, jy2) - jnp.maximum(iy1, jy1)
            marg = iw * ih - (ia3 + ja3)
            if masked:
                marg = jnp.where((j0 + jiota) < jlim, marg, -1.0)
            return jnp.maximum(acc, jnp.max(marg, axis=0, keepdims=True))

        nfull = jnp.minimum(i0, kk) // BJ
        acc0 = jnp.full((1, BI), -1.0, dtype=jnp.float32)
        acc = jax.lax.fori_loop(
            0, nfull, lambda jj, a: tile(jj * BJ, False, a), acc0)
        nall = (jnp.minimum(i0 + BI, kk) + (BJ - 1)) // BJ
        acc = jax.lax.fori_loop(
            nfull, nall, lambda jj, a: tile(jj * BJ, True, a), acc)

        keep = jnp.where((isc > SCORE_THR) & (acc <= 0.0), 1.0, 0.0)
        keep_ref[...] = keep.reshape(1, 1, BI)

    @pl.when(i0 >= kk)
    def _zeros():
        keep_ref[...] = jnp.zeros((1, 1, BI), dtype=jnp.float32)


@jax.jit
def kernel(x, guided_anchors):
    pad = NP - N
    s = jax.nn.sigmoid(x)
    kcount = jnp.sum(s > SCORE_THR).astype(jnp.int32).reshape(1)

    idx = jnp.arange(N, dtype=jnp.int32)
    cols = [guided_anchors[:, c] for c in range(7)]
    srt = jax.lax.sort([-s] + cols + [s, idx], num_keys=1, is_stable=True)
    order = srt[9]
    aT = jnp.stack(srt[1:9], axis=0)
    aT = jnp.pad(aT, ((0, 0), (0, pad)))

    keep_sorted = pl.pallas_call(
        _nms_body,
        grid=(NP // BI,),
        in_specs=[
            pl.BlockSpec(memory_space=pltpu.SMEM),
            pl.BlockSpec((8, NP), lambda i: (0, 0)),
        ],
        out_specs=pl.BlockSpec((1, 1, BI), lambda i: (i, 0, 0)),
        out_shape=jax.ShapeDtypeStruct((NP // BI, 1, BI), jnp.float32),
        scratch_shapes=[
            pltpu.VMEM((8, NP), jnp.float32),
            pltpu.VMEM((NP, 8), jnp.float32),
        ],
    )(kcount, aT)

    keep = jnp.zeros((N,), jnp.float32).at[order].set(
        keep_sorted.reshape(NP)[:N])
    return jnp.concatenate([guided_anchors, s[:, None]], axis=1) * keep[:, None]


# speedup vs baseline: 1.8598x; 1.3294x over previous
import jax
import jax.numpy as jnp
from jax.experimental import pallas as pl
from jax.experimental.pallas import tpu as pltpu

N = 5000
NP = 5120
BI = 1024
BJ = 1024
SCORE_THR = 0.3
THIRD = 1.0 / 3.0


def _nms_body(k_ref, aT_ref, keep_ref, fR_ref, fC_ref):
    i = pl.program_id(0)
    i0 = i * BI
    kk = k_ref[0]

    @pl.when(i == 0)
    def _features():
        cx = aT_ref[0:1, :]
        cy = aT_ref[1:2, :]
        w = aT_ref[3:4, :]
        l = aT_ref[4:5, :]
        ry = aT_ref[6:7, :]
        sc = aT_ref[7:8, :]
        c = jnp.abs(jnp.cos(ry))
        s = jnp.abs(jnp.sin(ry))
        hx = 0.5 * (w * c + l * s)
        hy = 0.5 * (w * s + l * c)
        x1 = cx - hx
        y1 = cy - hy
        x2 = cx + hx
        y2 = cy + hy
        a3 = ((x2 - x1) * (y2 - y1)) * THIRD
        rows = jnp.concatenate([x1, y1, x2, y2, a3, sc, sc, sc], axis=0)
        fR_ref[...] = rows
        fC_ref[...] = rows.T

    ix1 = fR_ref[0:1, pl.ds(i0, BI)]
    iy1 = fR_ref[1:2, pl.ds(i0, BI)]
    ix2 = fR_ref[2:3, pl.ds(i0, BI)]
    iy2 = fR_ref[3:4, pl.ds(i0, BI)]
    ia3 = fR_ref[4:5, pl.ds(i0, BI)]
    isc = fR_ref[5:6, pl.ds(i0, BI)]

    @pl.when(i0 < kk)
    def _compute():
        irank = i0 + jax.lax.broadcasted_iota(jnp.int32, (1, BI), 1)
        jlim = jnp.minimum(irank, kk)
        jiota = jax.lax.broadcasted_iota(jnp.int32, (BJ, 1), 0)

        def tile(j0, masked, acc):
            fj = fC_ref[pl.ds(j0, BJ), :]
            jx1 = fj[:, 0:1]
            jy1 = fj[:, 1:2]
            jx2 = fj[:, 2:3]
            jy2 = fj[:, 3:4]
            ja3 = fj[:, 4:5]

            iw = jnp.maximum(jnp.minimum(ix2, jx2) - jnp.maximum(ix1, jx1),
                             0.0)
            ih = jnp.minimum(iy2, jy2) - jnp.maximum(iy1, jy1)
            marg = iw * ih - (ia3 + ja3)
            if masked:
                marg = jnp.where((j0 + jiota) < jlim, marg, -1.0)
            return jnp.maximum(acc, jnp.max(marg, axis=0, keepdims=True))

        nfull = jnp.minimum(i0, kk) // BJ
        acc0 = jnp.full((1, BI), -1.0, dtype=jnp.float32)
        acc = jax.lax.fori_loop(
            0, nfull, lambda jj, a: tile(jj * BJ, False, a), acc0)
        nall = (jnp.minimum(i0 + BI, kk) + (BJ - 1)) // BJ
        acc = jax.lax.fori_loop(
            nfull, nall, lambda jj, a: tile(jj * BJ, True, a), acc)

        keep = jnp.where((isc > SCORE_THR) & (acc <= 0.0), 1.0, 0.0)
        keep_ref[...] = keep.reshape(1, 1, BI)

    @pl.when(i0 >= kk)
    def _zeros():
        keep_ref[...] = jnp.zeros((1, 1, BI), dtype=jnp.float32)


@jax.jit
def kernel(x, guided_anchors):
    pad = NP - N
    s = jax.nn.sigmoid(x)
    kcount = jnp.sum(s > SCORE_THR).astype(jnp.int32).reshape(1)

    idx = jnp.arange(N, dtype=jnp.int32)
    cols = [guided_anchors[:, c] for c in range(7)]
    srt = jax.lax.sort([-s] + cols + [s, idx], num_keys=1, is_stable=True)
    order = srt[9]
    aT = jnp.stack(srt[1:9], axis=0)
    aT = jnp.pad(aT, ((0, 0), (0, pad)))

    keep_sorted = pl.pallas_call(
        _nms_body,
        grid=(NP // BI,),
        in_specs=[
            pl.BlockSpec(memory_space=pltpu.SMEM),
            pl.BlockSpec((8, NP), lambda i: (0, 0)),
        ],
        out_specs=pl.BlockSpec((1, 1, BI), lambda i: (i, 0, 0)),
        out_shape=jax.ShapeDtypeStruct((NP // BI, 1, BI), jnp.float32),
        scratch_shapes=[
            pltpu.VMEM((8, NP), jnp.float32),
            pltpu.VMEM((NP, 8), jnp.float32),
        ],
    )(kcount, aT)

    keep = jax.lax.sort([order, keep_sorted.reshape(NP)[:N]], num_keys=1)[1]
    return jnp.concatenate([guided_anchors, s[:, None]], axis=1) * keep[:, None]


# speedup vs baseline: 1.9134x; 1.0288x over previous
import jax
import jax.numpy as jnp
from jax.experimental import pallas as pl
from jax.experimental.pallas import tpu as pltpu

N = 5000
NP = 5120
BI = 1024
BJ = 1024
SCORE_THR = 0.3
THIRD = 1.0 / 3.0


def _nms_body(k_ref, aT_ref, keep_ref, fR_ref, fC_ref):
    i = pl.program_id(0)
    i0 = i * BI
    kk = k_ref[0]

    @pl.when(i == 0)
    def _features():
        cx = aT_ref[0:1, :]
        cy = aT_ref[1:2, :]
        w = aT_ref[2:3, :]
        l = aT_ref[3:4, :]
        ry = aT_ref[4:5, :]
        sc = aT_ref[5:6, :]
        c = jnp.abs(jnp.cos(ry))
        s = jnp.abs(jnp.sin(ry))
        hx = 0.5 * (w * c + l * s)
        hy = 0.5 * (w * s + l * c)
        x1 = cx - hx
        y1 = cy - hy
        x2 = cx + hx
        y2 = cy + hy
        a3 = ((x2 - x1) * (y2 - y1)) * THIRD
        rows = jnp.concatenate([x1, y1, x2, y2, a3, sc, sc, sc], axis=0)
        fR_ref[...] = rows
        fC_ref[...] = rows.T

    ix1 = fR_ref[0:1, pl.ds(i0, BI)]
    iy1 = fR_ref[1:2, pl.ds(i0, BI)]
    ix2 = fR_ref[2:3, pl.ds(i0, BI)]
    iy2 = fR_ref[3:4, pl.ds(i0, BI)]
    ia3 = fR_ref[4:5, pl.ds(i0, BI)]
    isc = fR_ref[5:6, pl.ds(i0, BI)]

    @pl.when(i0 < kk)
    def _compute():
        irank = i0 + jax.lax.broadcasted_iota(jnp.int32, (1, BI), 1)
        jlim = jnp.minimum(irank, kk)
        jiota = jax.lax.broadcasted_iota(jnp.int32, (BJ, 1), 0)

        def tile(j0, masked, acc):
            fj = fC_ref[pl.ds(j0, BJ), :]
            jx1 = fj[:, 0:1]
            jy1 = fj[:, 1:2]
            jx2 = fj[:, 2:3]
            jy2 = fj[:, 3:4]
            ja3 = fj[:, 4:5]

            iw = jnp.maximum(jnp.minimum(ix2, jx2) - jnp.maximum(ix1, jx1),
                             0.0)
            ih = jnp.minimum(iy2, jy2) - jnp.maximum(iy1, jy1)
            marg = iw * ih - (ia3 + ja3)
            if masked:
                marg = jnp.where((j0 + jiota) < jlim, marg, -1.0)
            return jnp.maximum(acc, jnp.max(marg, axis=0, keepdims=True))

        nfull = jnp.minimum(i0, kk) // BJ
        acc0 = jnp.full((1, BI), -1.0, dtype=jnp.float32)
        acc = jax.lax.fori_loop(
            0, nfull, lambda jj, a: tile(jj * BJ, False, a), acc0)
        nall = (jnp.minimum(i0 + BI, kk) + (BJ - 1)) // BJ
        acc = jax.lax.fori_loop(
            nfull, nall, lambda jj, a: tile(jj * BJ, True, a), acc)

        keep = jnp.where((isc > SCORE_THR) & (acc <= 0.0), 1.0, 0.0)
        keep_ref[...] = keep.reshape(1, 1, BI)

    @pl.when(i0 >= kk)
    def _zeros():
        keep_ref[...] = jnp.zeros((1, 1, BI), dtype=jnp.float32)


@jax.jit
def kernel(x, guided_anchors):
    pad = NP - N
    s = jax.nn.sigmoid(x)
    kcount = jnp.sum(s > SCORE_THR).astype(jnp.int32).reshape(1)

    idx = jnp.arange(N, dtype=jnp.int32)
    cols = [guided_anchors[:, c] for c in (0, 1, 3, 4, 6)]
    srt = jax.lax.sort([-s] + cols + [s, idx], num_keys=1, is_stable=True)
    order = srt[7]
    aT = jnp.stack(srt[1:7], axis=0)
    aT = jnp.pad(aT, ((0, 2), (0, pad)))

    keep_sorted = pl.pallas_call(
        _nms_body,
        grid=(NP // BI,),
        in_specs=[
            pl.BlockSpec(memory_space=pltpu.SMEM),
            pl.BlockSpec((8, NP), lambda i: (0, 0)),
        ],
        out_specs=pl.BlockSpec((1, 1, BI), lambda i: (i, 0, 0)),
        out_shape=jax.ShapeDtypeStruct((NP // BI, 1, BI), jnp.float32),
        scratch_shapes=[
            pltpu.VMEM((8, NP), jnp.float32),
            pltpu.VMEM((NP, 8), jnp.float32),
        ],
    )(kcount, aT)

    keep = jax.lax.sort([order, keep_sorted.reshape(NP)[:N]], num_keys=1)[1]
    return jnp.concatenate([guided_anchors, s[:, None]], axis=1) * keep[:, None]
